# baseline (speedup 1.0000x reference)
import functools

import jax
import jax.numpy as jnp
from jax import lax
from jax.experimental import pallas as pl
from jax.experimental.pallas import tpu as pltpu
from jax.experimental.pallas import tpu_sc as plsc

NC = 2
NS = 16
NW = NC * NS
L = 16
NA = 8
DEPTH = 6


def _delta_body(t_ref, d_ref):
    x = t_ref[...]
    shifted = jnp.concatenate([x[:, :1], x[:, :-1]], axis=1)
    d_ref[...] = x - shifted


def _make_delta(B, T):
    return pl.pallas_call(
        _delta_body,
        out_shape=jax.ShapeDtypeStruct((B, T), jnp.float32),
    )


def _make_remap(B, T, V):
    n_act = NC * NA
    rows_w = B // n_act
    n_pass = 32
    rows_c = rows_w // n_pass
    chunk = rows_c * T
    n_vec = chunk // L

    mesh = plsc.VectorSubcoreMesh(core_axis_name="c", subcore_axis_name="s")

    @functools.partial(
        pl.kernel,
        out_type=jax.ShapeDtypeStruct((B, T), jnp.int32),
        mesh=mesh,
        compiler_params=pltpu.CompilerParams(needs_layout_passes=False),
        scratch_types=(
            [pltpu.VMEM((V,), jnp.int32)]
            + [pltpu.VMEM((rows_c, T), jnp.int32)] * DEPTH
            + [pltpu.VMEM((rows_c, T), jnp.int32)] * DEPTH
            + [pltpu.SemaphoreType.DMA] * (1 + 2 * DEPTH)
        ),
    )
    def k(cat_hbm, map_hbm, mp_hbm, map_v, *rest):
        cat_v = list(rest[0:DEPTH])
        mp_v = list(rest[DEPTH:2 * DEPTH])
        sem_t = rest[2 * DEPTH]
        sem_in = list(rest[2 * DEPTH + 1: 2 * DEPTH + 1 + DEPTH])
        sem_out = list(rest[2 * DEPTH + 1 + DEPTH:])
        sid = lax.axis_index("s")
        cid = lax.axis_index("c")
        wid = cid * NA + sid
        w_row = wid * rows_w

        def _remap_body():
            t_desc = pltpu.async_copy(map_hbm, map_v, sem_t)

            def start_in(pi):
                r0 = w_row + pi * rows_c
                s = pi % DEPTH
                return pltpu.async_copy(cat_hbm.at[pl.ds(r0, rows_c), :],
                                        cat_v[s], sem_in[s])

            in_descs = [None] * DEPTH
            out_descs = [None] * DEPTH
            for pi in range(DEPTH):
                in_descs[pi] = start_in(pi)
            lanes = lax.iota(jnp.int32, L)

            for pi in range(n_pass):
                s = pi % DEPTH
                in_descs[s].wait()
                if pi == 0:
                    t_desc.wait()
                if out_descs[s] is not None:
                    out_descs[s].wait()
                cv, mv = cat_v[s], mp_v[s]

                @plsc.parallel_loop(0, n_vec, 1, unroll=8)
                def body(j, cv=cv, mv=mv):
                    p = j * L + lanes
                    r = lax.div(p, T)
                    c = lax.rem(p, T)
                    ci = plsc.load_gather(cv, [r, c])
                    plsc.store_scatter(mv, [r, c], plsc.load_gather(map_v, [ci]))

                r0 = w_row + pi * rows_c
                out_descs[s] = pltpu.async_copy(
                    mv, mp_hbm.at[pl.ds(r0, rows_c), :], sem_out[s])
                if pi + DEPTH < n_pass:
                    in_descs[s] = start_in(pi + DEPTH)

            for s in range(DEPTH):
                if out_descs[s] is not None:
                    out_descs[s].wait()

        pl.when(sid < NA)(_remap_body)

    return k


def kernel(times, cat, mapping):
    B, T = times.shape
    V = mapping.shape[0]
    mapped = _make_remap(B, T, V)(cat, mapping)
    deltas = _make_delta(B, T)(times)
    return deltas, mapped


# speedup vs baseline: 1.0980x; 1.0980x over previous
import functools

import jax
import jax.numpy as jnp
from jax import lax
from jax.experimental import pallas as pl
from jax.experimental.pallas import tpu as pltpu
from jax.experimental.pallas import tpu_sc as plsc

NC = 2
NS = 16
NW = NC * NS
L = 16
DEPTH = 3


def _delta_body(t_ref, d_ref):
    x = t_ref[...]
    shifted = jnp.concatenate([x[:, :1], x[:, :-1]], axis=1)
    d_ref[...] = x - shifted


def _make_delta(B, T):
    return pl.pallas_call(
        _delta_body,
        out_shape=jax.ShapeDtypeStruct((B, T), jnp.float32),
    )


def _make_remap(B, T, V):
    rows_w = B // NW
    n_pass = 8
    rows_c = rows_w // n_pass
    chunk = rows_c * T
    n_vec = chunk // L

    mesh = plsc.VectorSubcoreMesh(core_axis_name="c", subcore_axis_name="s")

    @functools.partial(
        pl.kernel,
        out_type=jax.ShapeDtypeStruct((B, T), jnp.int32),
        mesh=mesh,
        compiler_params=pltpu.CompilerParams(needs_layout_passes=False),
        scratch_types=(
            [pltpu.VMEM((V,), jnp.int32)]
            + [pltpu.VMEM((rows_c, T), jnp.int32)] * DEPTH
            + [pltpu.VMEM((rows_c, T), jnp.int32)] * DEPTH
            + [pltpu.SemaphoreType.DMA] * (1 + 2 * DEPTH)
        ),
    )
    def k(cat_hbm, map_hbm, mp_hbm, map_v, *rest):
        cat_v = list(rest[0:DEPTH])
        mp_v = list(rest[DEPTH:2 * DEPTH])
        sem_t = rest[2 * DEPTH]
        sem_in = list(rest[2 * DEPTH + 1: 2 * DEPTH + 1 + DEPTH])
        sem_out = list(rest[2 * DEPTH + 1 + DEPTH:])
        wid = lax.axis_index("s") * NC + lax.axis_index("c")
        w_row = wid * rows_w

        t_desc = pltpu.async_copy(map_hbm, map_v, sem_t)

        def start_in(pi):
            r0 = w_row + pi * rows_c
            s = pi % DEPTH
            return pltpu.async_copy(cat_hbm.at[pl.ds(r0, rows_c), :],
                                    cat_v[s], sem_in[s])

        in_descs = [None] * DEPTH
        out_descs = [None] * DEPTH
        for pi in range(DEPTH):
            in_descs[pi] = start_in(pi)
        lanes = lax.iota(jnp.int32, L)

        for pi in range(n_pass):
            s = pi % DEPTH
            in_descs[s].wait()
            if pi == 0:
                t_desc.wait()
            if out_descs[s] is not None:
                out_descs[s].wait()
            cv, mv = cat_v[s], mp_v[s]

            @plsc.parallel_loop(0, n_vec, 1, unroll=8)
            def body(j, cv=cv, mv=mv):
                p = j * L + lanes
                r = lax.div(p, T)
                c = lax.rem(p, T)
                ci = plsc.load_gather(cv, [r, c])
                plsc.store_scatter(mv, [r, c], plsc.load_gather(map_v, [ci]))

            r0 = w_row + pi * rows_c
            out_descs[s] = pltpu.async_copy(
                mv, mp_hbm.at[pl.ds(r0, rows_c), :], sem_out[s])
            if pi + DEPTH < n_pass:
                in_descs[s] = start_in(pi + DEPTH)

        for s in range(DEPTH):
            if out_descs[s] is not None:
                out_descs[s].wait()

    return k


def kernel(times, cat, mapping):
    B, T = times.shape
    V = mapping.shape[0]
    mapped = _make_remap(B, T, V)(cat, mapping)
    deltas = _make_delta(B, T)(times)
    return deltas, mapped
